# baseline (speedup 1.0000x reference)
import functools

import jax
import jax.numpy as jnp
from jax import lax
from jax.experimental import pallas as pl
from jax.experimental.pallas import tpu as pltpu
from jax.experimental.pallas import tpu_sc as plsc

NU = 50000
NG = 10000
E = 320000
IN_DIM = 128
H = 2
D = 64
HD = H * D
ZC = 144
ERC = 8
NC = 2
NS = 16
NW = NC * NS
EPW = E // NW
CH = 80
NCHUNK = EPW // CH
NPAIR = (NCHUNK - 1) // 2
NGP = 10240
RPT = NGP // NS
LANES = 16


def _splat(v, j):
    idx = jnp.full((LANES, 1), j, jnp.int32)
    return lax.gather(
        v, idx,
        lax.GatherDimensionNumbers(
            offset_dims=(), collapsed_slice_dims=(0,), start_index_map=(0,)),
        (1,), mode=lax.GatherScatterMode.PROMISE_IN_BOUNDS)


def _edge_body(zext_r, er_r, src_r, dst_r, part_r,
               srcva, dstva, dstsa, zra, erba,
               srcvb, dstvb, dstsb, zrb, erbb,
               agg,
               zsa, esa, isa, ssa, zsb, esb, isb, ssb):
    c = lax.axis_index("c")
    s = lax.axis_index("s")
    wid = c * NS + s
    base = wid * EPW
    c128 = jnp.full((LANES,), HD, jnp.int32)
    c129 = jnp.full((LANES,), HD + 1, jnp.int32)
    col0 = jnp.zeros((LANES,), jnp.int32)
    col1 = jnp.ones((LANES,), jnp.int32)
    zv16 = jnp.zeros((LANES,), jnp.float32)

    def stage_idx(ci, srcv, dstv, isem):
        off = base + ci * CH
        pltpu.async_copy(src_r.at[pl.ds(off, CH)], srcv, isem)
        pltpu.async_copy(dst_r.at[pl.ds(off, CH)], dstv, isem)

    def wait_idx(srcv, dstv, isem):
        pltpu.make_async_copy(src_r.at[pl.ds(0, CH)], srcv, isem).wait()
        pltpu.make_async_copy(dst_r.at[pl.ds(0, CH)], dstv, isem).wait()

    CHH = CH // 2

    def issue_gathers(srcv, dstv, zr, erb, zsem, esem):
        pltpu.async_copy(zext_r.at[srcv.at[pl.ds(0, CHH)]],
                         zr.at[pl.ds(0, CHH)], zsem)
        pltpu.async_copy(zext_r.at[srcv.at[pl.ds(CHH, CHH)]],
                         zr.at[pl.ds(CHH, CHH)], esem)
        pltpu.async_copy(er_r.at[dstv], erb, zsem)

    def wait_gathers(srcv, dstv, zr, erb, zsem, esem):
        pltpu.make_async_copy(zext_r.at[srcv.at[pl.ds(0, CHH)]],
                              zr.at[pl.ds(0, CHH)], zsem).wait()
        pltpu.make_async_copy(zext_r.at[srcv.at[pl.ds(CHH, CHH)]],
                              zr.at[pl.ds(CHH, CHH)], esem).wait()
        pltpu.make_async_copy(er_r.at[dstv], erb, zsem).wait()

    def issue_scatter(zr, dsts, ssem):
        pltpu.async_copy(zr, agg.at[dsts], ssem, add=True)

    def wait_scatter(zr, dsts, ssem):
        pltpu.make_async_copy(zr, agg.at[dsts], ssem).wait()

    def compute_chunk(zr, erb, dstv, dsts):
        for i in range(CH // LANES):
            rows = lax.iota(jnp.int32, LANES) + (i * LANES)
            el0 = plsc.load_gather(zr, [rows, c128])
            el1 = plsc.load_gather(zr, [rows, c129])
            er0 = plsc.load_gather(erb, [rows, col0])
            er1 = plsc.load_gather(erb, [rows, col1])
            e0 = el0 + er0
            e1 = el1 + er1
            e0 = jnp.where(e0 >= 0.0, e0, e0 * 0.2)
            e1 = jnp.where(e1 >= 0.0, e1, e1 * 0.2)
            ex0 = jnp.exp(e0)
            ex1 = jnp.exp(e1)
            plsc.store_scatter(zr, [rows, c128], ex0)
            plsc.store_scatter(zr, [rows, c129], ex1)
            dsts[pl.ds(i * LANES, LANES)] = dstv[pl.ds(i * LANES, LANES)]
            for j in range(LANES):
                e_row = i * LANES + j
                sp0 = _splat(ex0, j)
                sp1 = _splat(ex1, j)
                for k in range(HD // LANES):
                    sp = sp0 if k < (D // LANES) else sp1
                    v = zr[e_row, pl.ds(k * LANES, LANES)]
                    zr[e_row, pl.ds(k * LANES, LANES)] = v * sp

    def zrow(r, carry):
        for kc in range(0, ZC, LANES):
            zra[r, pl.ds(kc, LANES)] = zv16
        return carry
    lax.fori_loop(0, CH, zrow, 0)
    for b in range(RPT // CH):
        pltpu.sync_copy(zra, agg.at[pl.ds(s * RPT + b * CH, CH)])
    plsc.subcore_barrier()

    stage_idx(0, srcva, dstva, isa)
    wait_idx(srcva, dstva, isa)
    issue_gathers(srcva, dstva, zra, erba, zsa, esa)

    def pair(t, carry):
        ci = 2 * t
        stage_idx(ci + 1, srcvb, dstvb, isb)
        wait_gathers(srcva, dstva, zra, erba, zsa, esa)
        wait_idx(srcvb, dstvb, isb)

        @pl.when(t > 0)
        def _():
            wait_scatter(zrb, dstsb, ssb)

        issue_gathers(srcvb, dstvb, zrb, erbb, zsb, esb)
        compute_chunk(zra, erba, dstva, dstsa)
        issue_scatter(zra, dstsa, ssa)
        stage_idx(ci + 2, srcva, dstva, isa)
        wait_gathers(srcvb, dstvb, zrb, erbb, zsb, esb)
        wait_idx(srcva, dstva, isa)
        wait_scatter(zra, dstsa, ssa)
        issue_gathers(srcva, dstva, zra, erba, zsa, esa)
        compute_chunk(zrb, erbb, dstvb, dstsb)
        issue_scatter(zrb, dstsb, ssb)
        return carry

    lax.fori_loop(0, NPAIR, pair, 0)

    wait_gathers(srcva, dstva, zra, erba, zsa, esa)
    wait_scatter(zrb, dstsb, ssb)
    compute_chunk(zra, erba, dstva, dstsa)
    issue_scatter(zra, dstsa, ssa)
    wait_scatter(zra, dstsa, ssa)

    plsc.subcore_barrier()
    pltpu.sync_copy(agg.at[pl.ds(s * RPT, RPT)],
                    part_r.at[c, pl.ds(s * RPT, RPT)])


@functools.cache
def _edge_phase():
    mesh = plsc.VectorSubcoreMesh(core_axis_name="c", subcore_axis_name="s")
    idx_t = pltpu.VMEM((CH,), jnp.int32)
    zr_t = pltpu.VMEM((CH, ZC), jnp.float32)
    erb_t = pltpu.VMEM((CH, ERC), jnp.float32)
    return pl.kernel(
        _edge_body,
        mesh=mesh,
        out_type=jax.ShapeDtypeStruct((NC, NGP, ZC), jnp.float32),
        scratch_types=[
            idx_t, idx_t, idx_t, zr_t, erb_t,
            idx_t, idx_t, idx_t, zr_t, erb_t,
            pltpu.VMEM_SHARED((NGP, ZC), jnp.float32),
            pltpu.SemaphoreType.DMA, pltpu.SemaphoreType.DMA,
            pltpu.SemaphoreType.DMA, pltpu.SemaphoreType.DMA,
            pltpu.SemaphoreType.DMA, pltpu.SemaphoreType.DMA,
            pltpu.SemaphoreType.DMA, pltpu.SemaphoreType.DMA,
        ],
        compiler_params=pltpu.CompilerParams(
            use_tc_tiling_on_sc=False, needs_layout_passes=False),
    )


def _matmul_body(x_ref, w_ref, o_ref):
    o_ref[...] = jnp.dot(x_ref[...], w_ref[...],
                         preferred_element_type=jnp.float32)


def _project(x, w, bm):
    m = x.shape[0]
    n = w.shape[1]
    return pl.pallas_call(
        _matmul_body,
        grid=(m // bm,),
        in_specs=[pl.BlockSpec((bm, IN_DIM), lambda i: (i, 0)),
                  pl.BlockSpec((IN_DIM, n), lambda i: (0, 0))],
        out_specs=pl.BlockSpec((bm, n), lambda i: (i, 0)),
        out_shape=jax.ShapeDtypeStruct((m, n), jnp.float32),
    )(x, w)


def _finish_body(p_ref, o_ref):
    num = p_ref[0] + p_ref[1]
    x = num[:, :HD]
    s0 = num[:, HD:HD + 1]
    s1 = num[:, HD + 1:HD + 2]
    col = lax.broadcasted_iota(jnp.int32, x.shape, 1)
    den = jnp.where(col < D, s0, s1) + 1e-9
    r = x / den
    o_ref[...] = jnp.where(r > 0.0, r, jnp.exp(r) - 1.0)


def _finish(part, bm):
    return pl.pallas_call(
        _finish_body,
        grid=(NG // bm,),
        in_specs=[pl.BlockSpec((NC, bm, ZC), lambda i: (0, i, 0))],
        out_specs=pl.BlockSpec((bm, HD), lambda i: (i, 0)),
        out_shape=jax.ShapeDtypeStruct((NG, HD), jnp.float32),
    )(part)


def kernel(user_emb, group_emb, W, attn_l, attn_r, src, dst):
    src = src.astype(jnp.int32)
    dst = dst.astype(jnp.int32)
    w3 = W.reshape(IN_DIM, H, D)
    wl = jnp.einsum("ihd,hd->ih", w3, attn_l)
    wr = jnp.einsum("ihd,hd->ih", w3, attn_r)
    w_aug = jnp.concatenate(
        [W, wl, jnp.zeros((IN_DIM, ZC - HD - H), jnp.float32)], axis=1)
    zext = _project(user_emb, w_aug, 1000)
    wr_pad = jnp.concatenate(
        [wr, jnp.zeros((IN_DIM, ERC - H), jnp.float32)], axis=1)
    er = _project(group_emb, wr_pad, 1000)
    part = _edge_phase()(zext, er, src, dst)
    return _finish(part, 1000)


# speedup vs baseline: 1.0295x; 1.0295x over previous
import functools

import jax
import jax.numpy as jnp
from jax import lax
from jax.experimental import pallas as pl
from jax.experimental.pallas import tpu as pltpu
from jax.experimental.pallas import tpu_sc as plsc

NU = 50000
NG = 10000
E = 320000
IN_DIM = 128
H = 2
D = 64
HD = H * D
ZC = 144
ERC = 8
NC = 2
NS = 16
NW = NC * NS
EPW = E // NW
CH = 80
NCHUNK = EPW // CH
NPAIR = (NCHUNK - 1) // 2
NGP = 10240
RPT = NGP // NS
LANES = 16


def _splat(v, j):
    idx = jnp.full((LANES, 1), j, jnp.int32)
    return lax.gather(
        v, idx,
        lax.GatherDimensionNumbers(
            offset_dims=(), collapsed_slice_dims=(0,), start_index_map=(0,)),
        (1,), mode=lax.GatherScatterMode.PROMISE_IN_BOUNDS)


def _edge_body(zext_r, er_r, src_r, dst_r, part_r,
               srcva, dstva, dstsa, zra, erba,
               srcvb, dstvb, dstsb, zrb, erbb,
               agg,
               zsa, esa, isa, ssa, zsb, esb, isb, ssb):
    c = lax.axis_index("c")
    s = lax.axis_index("s")
    wid = c * NS + s
    base = wid * EPW
    c128 = jnp.full((LANES,), HD, jnp.int32)
    c129 = jnp.full((LANES,), HD + 1, jnp.int32)
    col0 = jnp.zeros((LANES,), jnp.int32)
    col1 = jnp.ones((LANES,), jnp.int32)
    zv16 = jnp.zeros((LANES,), jnp.float32)

    def stage_idx(ci, srcv, dstv, isem):
        off = base + ci * CH
        pltpu.async_copy(src_r.at[pl.ds(off, CH)], srcv, isem)
        pltpu.async_copy(dst_r.at[pl.ds(off, CH)], dstv, isem)

    def wait_idx(srcv, dstv, isem):
        pltpu.make_async_copy(src_r.at[pl.ds(0, CH)], srcv, isem).wait()
        pltpu.make_async_copy(dst_r.at[pl.ds(0, CH)], dstv, isem).wait()

    def issue_gathers(srcv, dstv, zr, erb, zsem, esem):
        pltpu.async_copy(zext_r.at[srcv], zr, zsem)
        pltpu.async_copy(er_r.at[dstv], erb, esem)

    def wait_gathers(srcv, dstv, zr, erb, zsem, esem):
        pltpu.make_async_copy(zext_r.at[srcv], zr, zsem).wait()
        pltpu.make_async_copy(er_r.at[dstv], erb, esem).wait()

    def issue_scatter(zr, dsts, ssem):
        pltpu.async_copy(zr, agg.at[dsts], ssem, add=True)

    def wait_scatter(zr, dsts, ssem):
        pltpu.make_async_copy(zr, agg.at[dsts], ssem).wait()

    def compute_chunk(zr, erb, dstv, dsts):
        for i in range(CH // LANES):
            rows = lax.iota(jnp.int32, LANES) + (i * LANES)
            el0 = plsc.load_gather(zr, [rows, c128])
            el1 = plsc.load_gather(zr, [rows, c129])
            er0 = plsc.load_gather(erb, [rows, col0])
            er1 = plsc.load_gather(erb, [rows, col1])
            e0 = el0 + er0
            e1 = el1 + er1
            e0 = jnp.where(e0 >= 0.0, e0, e0 * 0.2)
            e1 = jnp.where(e1 >= 0.0, e1, e1 * 0.2)
            ex0 = jnp.exp(e0)
            ex1 = jnp.exp(e1)
            plsc.store_scatter(zr, [rows, c128], ex0)
            plsc.store_scatter(zr, [rows, c129], ex1)
            dsts[pl.ds(i * LANES, LANES)] = dstv[pl.ds(i * LANES, LANES)]
            for j in range(LANES):
                e_row = i * LANES + j
                sp0 = _splat(ex0, j)
                sp1 = _splat(ex1, j)
                for k in range(HD // LANES):
                    sp = sp0 if k < (D // LANES) else sp1
                    v = zr[e_row, pl.ds(k * LANES, LANES)]
                    zr[e_row, pl.ds(k * LANES, LANES)] = v * sp

    def zrow(r, carry):
        for kc in range(0, ZC, LANES):
            zra[r, pl.ds(kc, LANES)] = zv16
        return carry
    lax.fori_loop(0, CH, zrow, 0)
    for b in range(RPT // CH):
        pltpu.sync_copy(zra, agg.at[pl.ds(s * RPT + b * CH, CH)])
    plsc.subcore_barrier()

    stage_idx(0, srcva, dstva, isa)
    wait_idx(srcva, dstva, isa)
    issue_gathers(srcva, dstva, zra, erba, zsa, esa)

    def pair(t, carry):
        ci = 2 * t
        stage_idx(ci + 1, srcvb, dstvb, isb)
        wait_gathers(srcva, dstva, zra, erba, zsa, esa)
        wait_idx(srcvb, dstvb, isb)

        @pl.when(t > 0)
        def _():
            wait_scatter(zrb, dstsb, ssb)

        issue_gathers(srcvb, dstvb, zrb, erbb, zsb, esb)
        compute_chunk(zra, erba, dstva, dstsa)
        issue_scatter(zra, dstsa, ssa)
        stage_idx(ci + 2, srcva, dstva, isa)
        wait_gathers(srcvb, dstvb, zrb, erbb, zsb, esb)
        wait_idx(srcva, dstva, isa)
        wait_scatter(zra, dstsa, ssa)
        issue_gathers(srcva, dstva, zra, erba, zsa, esa)
        compute_chunk(zrb, erbb, dstvb, dstsb)
        issue_scatter(zrb, dstsb, ssb)
        return carry

    lax.fori_loop(0, NPAIR, pair, 0)

    wait_gathers(srcva, dstva, zra, erba, zsa, esa)
    wait_scatter(zrb, dstsb, ssb)
    compute_chunk(zra, erba, dstva, dstsa)
    issue_scatter(zra, dstsa, ssa)
    wait_scatter(zra, dstsa, ssa)

    plsc.subcore_barrier()
    pltpu.sync_copy(agg.at[pl.ds(s * RPT, RPT)],
                    part_r.at[c, pl.ds(s * RPT, RPT)])


@functools.cache
def _edge_phase():
    mesh = plsc.VectorSubcoreMesh(core_axis_name="c", subcore_axis_name="s")
    idx_t = pltpu.VMEM((CH,), jnp.int32)
    zr_t = pltpu.VMEM((CH, ZC), jnp.float32)
    erb_t = pltpu.VMEM((CH, ERC), jnp.float32)
    return pl.kernel(
        _edge_body,
        mesh=mesh,
        out_type=jax.ShapeDtypeStruct((NC, NGP, ZC), jnp.float32),
        scratch_types=[
            idx_t, idx_t, idx_t, zr_t, erb_t,
            idx_t, idx_t, idx_t, zr_t, erb_t,
            pltpu.VMEM_SHARED((NGP, ZC), jnp.float32),
            pltpu.SemaphoreType.DMA, pltpu.SemaphoreType.DMA,
            pltpu.SemaphoreType.DMA, pltpu.SemaphoreType.DMA,
            pltpu.SemaphoreType.DMA, pltpu.SemaphoreType.DMA,
            pltpu.SemaphoreType.DMA, pltpu.SemaphoreType.DMA,
        ],
        compiler_params=pltpu.CompilerParams(
            use_tc_tiling_on_sc=False, needs_layout_passes=False),
    )


def _proj2_body(x_ref, g_ref, w_ref, wr_ref, o_ref, e_ref):
    o_ref[...] = jnp.dot(x_ref[...], w_ref[...],
                         preferred_element_type=jnp.float32)

    @pl.when(pl.program_id(0) < NG // 1000)
    def _():
        e_ref[...] = jnp.dot(g_ref[...], wr_ref[...],
                             preferred_element_type=jnp.float32)


def _project2(x, g, w, wr, bm):
    gmin = NG // bm - 1

    def gmap(i):
        return (jnp.minimum(i, gmin), 0)

    return pl.pallas_call(
        _proj2_body,
        grid=(NU // bm,),
        in_specs=[pl.BlockSpec((bm, IN_DIM), lambda i: (i, 0)),
                  pl.BlockSpec((bm, IN_DIM), gmap),
                  pl.BlockSpec((IN_DIM, ZC), lambda i: (0, 0)),
                  pl.BlockSpec((IN_DIM, ERC), lambda i: (0, 0))],
        out_specs=[pl.BlockSpec((bm, ZC), lambda i: (i, 0)),
                   pl.BlockSpec((bm, ERC), gmap)],
        out_shape=[jax.ShapeDtypeStruct((NU, ZC), jnp.float32),
                   jax.ShapeDtypeStruct((NG, ERC), jnp.float32)],
    )(x, g, w, wr)


def _finish_body(p_ref, o_ref):
    num = p_ref[0] + p_ref[1]
    x = num[:, :HD]
    s0 = num[:, HD:HD + 1]
    s1 = num[:, HD + 1:HD + 2]
    col = lax.broadcasted_iota(jnp.int32, x.shape, 1)
    den = jnp.where(col < D, s0, s1) + 1e-9
    r = x / den
    o_ref[...] = jnp.where(r > 0.0, r, jnp.exp(r) - 1.0)


def _finish(part, bm):
    return pl.pallas_call(
        _finish_body,
        grid=(NG // bm,),
        in_specs=[pl.BlockSpec((NC, bm, ZC), lambda i: (0, i, 0))],
        out_specs=pl.BlockSpec((bm, HD), lambda i: (i, 0)),
        out_shape=jax.ShapeDtypeStruct((NG, HD), jnp.float32),
    )(part)


def kernel(user_emb, group_emb, W, attn_l, attn_r, src, dst):
    src = src.astype(jnp.int32)
    dst = dst.astype(jnp.int32)
    w3 = W.reshape(IN_DIM, H, D)
    wl = jnp.einsum("ihd,hd->ih", w3, attn_l)
    wr = jnp.einsum("ihd,hd->ih", w3, attn_r)
    w_aug = jnp.concatenate(
        [W, wl, jnp.zeros((IN_DIM, ZC - HD - H), jnp.float32)], axis=1)
    wr_pad = jnp.concatenate(
        [wr, jnp.zeros((IN_DIM, ERC - H), jnp.float32)], axis=1)
    zext, er = _project2(user_emb, group_emb, w_aug, wr_pad, 1000)
    part = _edge_phase()(zext, er, src, dst)
    return _finish(part, 1000)


# speedup vs baseline: 1.0742x; 1.0434x over previous
import functools

import jax
import jax.numpy as jnp
from jax import lax
from jax.experimental import pallas as pl
from jax.experimental.pallas import tpu as pltpu
from jax.experimental.pallas import tpu_sc as plsc

NU = 50000
NG = 10000
E = 320000
IN_DIM = 128
H = 2
D = 64
HD = H * D
ZC = 144
ERC = 8
NC = 2
NS = 16
NW = NC * NS
EPW = E // NW
CH = 80
NCHUNK = EPW // CH
NPAIR = (NCHUNK - 1) // 2
NGP = 10240
RPT = NGP // NS
LANES = 16


def _splat(v, j):
    idx = jnp.full((LANES, 1), j, jnp.int32)
    return lax.gather(
        v, idx,
        lax.GatherDimensionNumbers(
            offset_dims=(), collapsed_slice_dims=(0,), start_index_map=(0,)),
        (1,), mode=lax.GatherScatterMode.PROMISE_IN_BOUNDS)


def _edge_body(zext_r, er_r, src_r, dst_r, part_r,
               srcva, dstva, dstsa, zra, erba,
               srcvb, dstvb, dstsb, zrb, erbb,
               agg,
               zsa, esa, isa, ssa, zsb, esb, isb, ssb):
    c = lax.axis_index("c")
    s = lax.axis_index("s")
    wid = c * NS + s
    base = wid * EPW
    c128 = jnp.full((LANES,), HD, jnp.int32)
    c129 = jnp.full((LANES,), HD + 1, jnp.int32)
    col0 = jnp.zeros((LANES,), jnp.int32)
    col1 = jnp.ones((LANES,), jnp.int32)
    zv16 = jnp.zeros((LANES,), jnp.float32)

    def stage_idx(ci, srcv, dstv, isem):
        off = base + ci * CH
        pltpu.async_copy(src_r.at[pl.ds(off, CH)], srcv, isem)
        pltpu.async_copy(dst_r.at[pl.ds(off, CH)], dstv, isem)

    def wait_idx(srcv, dstv, isem):
        pltpu.make_async_copy(src_r.at[pl.ds(0, CH)], srcv, isem).wait()
        pltpu.make_async_copy(dst_r.at[pl.ds(0, CH)], dstv, isem).wait()

    def issue_gathers(srcv, dstv, zr, erb, zsem, esem):
        pltpu.async_copy(zext_r.at[srcv], zr, zsem)
        pltpu.async_copy(er_r.at[dstv], erb, esem)

    def wait_gathers(srcv, dstv, zr, erb, zsem, esem):
        pltpu.make_async_copy(zext_r.at[srcv], zr, zsem).wait()
        pltpu.make_async_copy(er_r.at[dstv], erb, esem).wait()

    def issue_scatter(zr, dsts, ssem):
        pltpu.async_copy(zr, agg.at[dsts], ssem, add=True)

    def wait_scatter(zr, dsts, ssem):
        pltpu.make_async_copy(zr, agg.at[dsts], ssem).wait()

    def compute_chunk(zr, erb, dstv, dsts):
        for i in range(CH // LANES):
            rows = lax.iota(jnp.int32, LANES) + (i * LANES)
            el0 = plsc.load_gather(zr, [rows, c128])
            el1 = plsc.load_gather(zr, [rows, c129])
            er0 = plsc.load_gather(erb, [rows, col0])
            er1 = plsc.load_gather(erb, [rows, col1])
            e0 = el0 + er0
            e1 = el1 + er1
            e0 = jnp.where(e0 >= 0.0, e0, e0 * 0.2)
            e1 = jnp.where(e1 >= 0.0, e1, e1 * 0.2)
            ex0 = jnp.exp(e0)
            ex1 = jnp.exp(e1)
            plsc.store_scatter(zr, [rows, c128], ex0)
            plsc.store_scatter(zr, [rows, c129], ex1)
            dsts[pl.ds(i * LANES, LANES)] = dstv[pl.ds(i * LANES, LANES)]
            for j in range(LANES):
                e_row = i * LANES + j
                sp0 = _splat(ex0, j)
                sp1 = _splat(ex1, j)
                for k in range(HD // LANES):
                    sp = sp0 if k < (D // LANES) else sp1
                    v = zr[e_row, pl.ds(k * LANES, LANES)]
                    zr[e_row, pl.ds(k * LANES, LANES)] = v * sp

    def zrow(r, carry):
        for kc in range(0, ZC, LANES):
            zra[r, pl.ds(kc, LANES)] = zv16
        return carry
    lax.fori_loop(0, CH, zrow, 0)
    for b in range(RPT // CH):
        pltpu.sync_copy(zra, agg.at[pl.ds(s * RPT + b * CH, CH)])
    plsc.subcore_barrier()

    stage_idx(0, srcva, dstva, isa)
    wait_idx(srcva, dstva, isa)
    issue_gathers(srcva, dstva, zra, erba, zsa, esa)

    def pair(t, carry):
        ci = 2 * t
        stage_idx(ci + 1, srcvb, dstvb, isb)
        wait_gathers(srcva, dstva, zra, erba, zsa, esa)
        wait_idx(srcvb, dstvb, isb)

        @pl.when(t > 0)
        def _():
            wait_scatter(zrb, dstsb, ssb)

        issue_gathers(srcvb, dstvb, zrb, erbb, zsb, esb)
        compute_chunk(zra, erba, dstva, dstsa)
        issue_scatter(zra, dstsa, ssa)
        stage_idx(ci + 2, srcva, dstva, isa)
        wait_gathers(srcvb, dstvb, zrb, erbb, zsb, esb)
        wait_idx(srcva, dstva, isa)
        wait_scatter(zra, dstsa, ssa)
        issue_gathers(srcva, dstva, zra, erba, zsa, esa)
        compute_chunk(zrb, erbb, dstvb, dstsb)
        issue_scatter(zrb, dstsb, ssb)
        return carry

    lax.fori_loop(0, NPAIR, pair, 0)

    wait_gathers(srcva, dstva, zra, erba, zsa, esa)
    wait_scatter(zrb, dstsb, ssb)
    compute_chunk(zra, erba, dstva, dstsa)
    issue_scatter(zra, dstsa, ssa)
    wait_scatter(zra, dstsa, ssa)

    plsc.subcore_barrier()
    pltpu.sync_copy(agg.at[pl.ds(s * RPT, RPT)],
                    part_r.at[c, pl.ds(s * RPT, RPT)])


@functools.cache
def _edge_phase():
    mesh = plsc.VectorSubcoreMesh(core_axis_name="c", subcore_axis_name="s")
    idx_t = pltpu.VMEM((CH,), jnp.int32)
    zr_t = pltpu.VMEM((CH, ZC), jnp.float32)
    erb_t = pltpu.VMEM((CH, ERC), jnp.float32)
    return pl.kernel(
        _edge_body,
        mesh=mesh,
        out_type=jax.ShapeDtypeStruct((NC, NGP, ZC), jnp.float32),
        scratch_types=[
            idx_t, idx_t, idx_t, zr_t, erb_t,
            idx_t, idx_t, idx_t, zr_t, erb_t,
            pltpu.VMEM_SHARED((NGP, ZC), jnp.float32),
            pltpu.SemaphoreType.DMA, pltpu.SemaphoreType.DMA,
            pltpu.SemaphoreType.DMA, pltpu.SemaphoreType.DMA,
            pltpu.SemaphoreType.DMA, pltpu.SemaphoreType.DMA,
            pltpu.SemaphoreType.DMA, pltpu.SemaphoreType.DMA,
        ],
        compiler_params=pltpu.CompilerParams(
            use_tc_tiling_on_sc=False, needs_layout_passes=False),
    )


def _proj2_body(x_ref, g_ref, w_ref, wr_ref, o_ref, e_ref):
    o_ref[...] = jnp.dot(x_ref[...], w_ref[...],
                         preferred_element_type=jnp.float32)

    @pl.when(pl.program_id(0) < NG // 1000)
    def _():
        e_ref[...] = jnp.dot(g_ref[...], wr_ref[...],
                             preferred_element_type=jnp.float32)


def _project2(x, g, w, wr, bm):
    gmin = max(NG // bm - 1, 0)

    def gmap(i):
        return (jnp.minimum(i, gmin), 0)

    return pl.pallas_call(
        _proj2_body,
        grid=(NU // bm,),
        in_specs=[pl.BlockSpec((bm, IN_DIM), lambda i: (i, 0)),
                  pl.BlockSpec((bm, IN_DIM), gmap),
                  pl.BlockSpec((IN_DIM, ZC), lambda i: (0, 0)),
                  pl.BlockSpec((IN_DIM, ERC), lambda i: (0, 0))],
        out_specs=[pl.BlockSpec((bm, ZC), lambda i: (i, 0)),
                   pl.BlockSpec((bm, ERC), gmap)],
        out_shape=[jax.ShapeDtypeStruct((NU, ZC), jnp.float32),
                   jax.ShapeDtypeStruct((NG, ERC), jnp.float32)],
    )(x, g, w, wr)


def _finish_body(p_ref, o_ref):
    num = p_ref[0] + p_ref[1]
    x = num[:, :HD]
    s0 = num[:, HD:HD + 1]
    s1 = num[:, HD + 1:HD + 2]
    col = lax.broadcasted_iota(jnp.int32, x.shape, 1)
    den = jnp.where(col < D, s0, s1) + 1e-9
    r = x / den
    o_ref[...] = jnp.where(r > 0.0, r, jnp.exp(r) - 1.0)


def _finish(part, bm):
    return pl.pallas_call(
        _finish_body,
        grid=(NG // bm,),
        in_specs=[pl.BlockSpec((NC, bm, ZC), lambda i: (0, i, 0))],
        out_specs=pl.BlockSpec((bm, HD), lambda i: (i, 0)),
        out_shape=jax.ShapeDtypeStruct((NG, HD), jnp.float32),
    )(part)


def kernel(user_emb, group_emb, W, attn_l, attn_r, src, dst):
    src = src.astype(jnp.int32)
    dst = dst.astype(jnp.int32)
    w3 = W.reshape(IN_DIM, H, D)
    wl = jnp.einsum("ihd,hd->ih", w3, attn_l)
    wr = jnp.einsum("ihd,hd->ih", w3, attn_r)
    w_aug = jnp.concatenate(
        [W, wl, jnp.zeros((IN_DIM, ZC - HD - H), jnp.float32)], axis=1)
    wr_pad = jnp.concatenate(
        [wr, jnp.zeros((IN_DIM, ERC - H), jnp.float32)], axis=1)
    zext, er = _project2(user_emb, group_emb, w_aug, wr_pad, 2000)
    part = _edge_phase()(zext, er, src, dst)
    return _finish(part, 2000)


# speedup vs baseline: 1.0946x; 1.0189x over previous
import functools

import jax
import jax.numpy as jnp
from jax import lax
from jax.experimental import pallas as pl
from jax.experimental.pallas import tpu as pltpu
from jax.experimental.pallas import tpu_sc as plsc

NU = 50000
NG = 10000
E = 320000
IN_DIM = 128
H = 2
D = 64
HD = H * D
ZC = 144
ERC = 8
NC = 2
NS = 16
NW = NC * NS
EPW = E // NW
CH = 80
NCHUNK = EPW // CH
NPAIR = (NCHUNK - 1) // 2
NGP = 10240
RPT = NGP // NS
LANES = 16


def _splat(v, j):
    idx = jnp.full((LANES, 1), j, jnp.int32)
    return lax.gather(
        v, idx,
        lax.GatherDimensionNumbers(
            offset_dims=(), collapsed_slice_dims=(0,), start_index_map=(0,)),
        (1,), mode=lax.GatherScatterMode.PROMISE_IN_BOUNDS)


def _edge_body(zext_r, er_r, src_r, dst_r, part_r,
               srcva, dstva, dstsa, zra, erba,
               srcvb, dstvb, dstsb, zrb, erbb,
               agg,
               zsa, esa, isa, ssa, zsb, esb, isb, ssb):
    c = lax.axis_index("c")
    s = lax.axis_index("s")
    wid = c * NS + s
    base = wid * EPW
    c128 = jnp.full((LANES,), HD, jnp.int32)
    c129 = jnp.full((LANES,), HD + 1, jnp.int32)
    col0 = jnp.zeros((LANES,), jnp.int32)
    col1 = jnp.ones((LANES,), jnp.int32)
    zv16 = jnp.zeros((LANES,), jnp.float32)

    def stage_idx(ci, srcv, dstv, isem):
        off = base + ci * CH
        pltpu.async_copy(src_r.at[pl.ds(off, CH)], srcv, isem)
        pltpu.async_copy(dst_r.at[pl.ds(off, CH)], dstv, isem)

    def wait_idx(srcv, dstv, isem):
        pltpu.make_async_copy(src_r.at[pl.ds(0, CH)], srcv, isem).wait()
        pltpu.make_async_copy(dst_r.at[pl.ds(0, CH)], dstv, isem).wait()

    def issue_gathers(srcv, dstv, zr, erb, zsem, esem):
        pltpu.async_copy(zext_r.at[srcv], zr, zsem)
        pltpu.async_copy(er_r.at[dstv], erb, esem)

    def wait_gathers(srcv, dstv, zr, erb, zsem, esem):
        pltpu.make_async_copy(zext_r.at[srcv], zr, zsem).wait()
        pltpu.make_async_copy(er_r.at[dstv], erb, esem).wait()

    def issue_scatter(zr, dsts, ssem):
        pltpu.async_copy(zr, agg.at[dsts], ssem, add=True)

    def wait_scatter(zr, dsts, ssem):
        pltpu.make_async_copy(zr, agg.at[dsts], ssem).wait()

    def compute_chunk(zr, erb, dstv, dsts):
        for i in range(CH // LANES):
            rows = lax.iota(jnp.int32, LANES) + (i * LANES)
            el0 = plsc.load_gather(zr, [rows, c128])
            el1 = plsc.load_gather(zr, [rows, c129])
            er0 = plsc.load_gather(erb, [rows, col0])
            er1 = plsc.load_gather(erb, [rows, col1])
            e0 = el0 + er0
            e1 = el1 + er1
            e0 = jnp.where(e0 >= 0.0, e0, e0 * 0.2)
            e1 = jnp.where(e1 >= 0.0, e1, e1 * 0.2)
            ex0 = jnp.exp(e0)
            ex1 = jnp.exp(e1)
            plsc.store_scatter(zr, [rows, c128], ex0)
            plsc.store_scatter(zr, [rows, c129], ex1)
            dsts[pl.ds(i * LANES, LANES)] = dstv[pl.ds(i * LANES, LANES)]
            for j in range(LANES):
                e_row = i * LANES + j
                sp0 = _splat(ex0, j)
                sp1 = _splat(ex1, j)
                for k in range(HD // LANES):
                    sp = sp0 if k < (D // LANES) else sp1
                    v = zr[e_row, pl.ds(k * LANES, LANES)]
                    zr[e_row, pl.ds(k * LANES, LANES)] = v * sp

    def zrow(r, carry):
        for kc in range(0, ZC, LANES):
            zra[r, pl.ds(kc, LANES)] = zv16
        return carry
    lax.fori_loop(0, CH, zrow, 0)
    for b in range(RPT // CH):
        pltpu.sync_copy(zra, agg.at[pl.ds(s * RPT + b * CH, CH)])
    plsc.subcore_barrier()

    stage_idx(0, srcva, dstva, isa)
    wait_idx(srcva, dstva, isa)
    issue_gathers(srcva, dstva, zra, erba, zsa, esa)

    def pair(t, carry):
        ci = 2 * t
        stage_idx(ci + 1, srcvb, dstvb, isb)
        wait_gathers(srcva, dstva, zra, erba, zsa, esa)
        wait_idx(srcvb, dstvb, isb)

        @pl.when(t > 0)
        def _():
            wait_scatter(zrb, dstsb, ssb)

        issue_gathers(srcvb, dstvb, zrb, erbb, zsb, esb)
        compute_chunk(zra, erba, dstva, dstsa)
        issue_scatter(zra, dstsa, ssa)
        stage_idx(ci + 2, srcva, dstva, isa)
        wait_gathers(srcvb, dstvb, zrb, erbb, zsb, esb)
        wait_idx(srcva, dstva, isa)
        wait_scatter(zra, dstsa, ssa)
        issue_gathers(srcva, dstva, zra, erba, zsa, esa)
        compute_chunk(zrb, erbb, dstvb, dstsb)
        issue_scatter(zrb, dstsb, ssb)
        return carry

    lax.fori_loop(0, NPAIR, pair, 0)

    wait_gathers(srcva, dstva, zra, erba, zsa, esa)
    wait_scatter(zrb, dstsb, ssb)
    compute_chunk(zra, erba, dstva, dstsa)
    issue_scatter(zra, dstsa, ssa)
    wait_scatter(zra, dstsa, ssa)

    plsc.subcore_barrier()
    pltpu.sync_copy(agg.at[pl.ds(s * RPT, RPT)],
                    part_r.at[c, pl.ds(s * RPT, RPT)])


@functools.cache
def _edge_phase():
    mesh = plsc.VectorSubcoreMesh(core_axis_name="c", subcore_axis_name="s")
    idx_t = pltpu.VMEM((CH,), jnp.int32)
    zr_t = pltpu.VMEM((CH, ZC), jnp.float32)
    erb_t = pltpu.VMEM((CH, ERC), jnp.float32)
    return pl.kernel(
        _edge_body,
        mesh=mesh,
        out_type=jax.ShapeDtypeStruct((NC, NGP, ZC), jnp.float32),
        scratch_types=[
            idx_t, idx_t, idx_t, zr_t, erb_t,
            idx_t, idx_t, idx_t, zr_t, erb_t,
            pltpu.VMEM_SHARED((NGP, ZC), jnp.float32),
            pltpu.SemaphoreType.DMA, pltpu.SemaphoreType.DMA,
            pltpu.SemaphoreType.DMA, pltpu.SemaphoreType.DMA,
            pltpu.SemaphoreType.DMA, pltpu.SemaphoreType.DMA,
            pltpu.SemaphoreType.DMA, pltpu.SemaphoreType.DMA,
        ],
        compiler_params=pltpu.CompilerParams(
            use_tc_tiling_on_sc=False, needs_layout_passes=False),
    )


def _proj2_body(x_ref, g_ref, w_ref, wr_ref, o_ref, e_ref):
    o_ref[...] = jnp.dot(x_ref[...], w_ref[...],
                         preferred_element_type=jnp.float32)

    @pl.when(pl.program_id(0) < NG // 1000)
    def _():
        e_ref[...] = jnp.dot(g_ref[...], wr_ref[...],
                             preferred_element_type=jnp.float32)


def _project2(x, g, w, wr, bm):
    gmin = max(NG // bm - 1, 0)

    def gmap(i):
        return (jnp.minimum(i, gmin), 0)

    return pl.pallas_call(
        _proj2_body,
        grid=(NU // bm,),
        in_specs=[pl.BlockSpec((bm, IN_DIM), lambda i: (i, 0)),
                  pl.BlockSpec((bm, IN_DIM), gmap),
                  pl.BlockSpec((IN_DIM, ZC), lambda i: (0, 0)),
                  pl.BlockSpec((IN_DIM, ERC), lambda i: (0, 0))],
        out_specs=[pl.BlockSpec((bm, ZC), lambda i: (i, 0)),
                   pl.BlockSpec((bm, ERC), gmap)],
        out_shape=[jax.ShapeDtypeStruct((NU, ZC), jnp.float32),
                   jax.ShapeDtypeStruct((NG, ERC), jnp.float32)],
    )(x, g, w, wr)


def _finish_body(p_ref, o_ref):
    num = p_ref[0] + p_ref[1]
    x = num[:, :HD]
    s0 = num[:, HD:HD + 1]
    s1 = num[:, HD + 1:HD + 2]
    col = lax.broadcasted_iota(jnp.int32, x.shape, 1)
    den = jnp.where(col < D, s0, s1) + 1e-9
    r = x / den
    o_ref[...] = jnp.where(r > 0.0, r, jnp.exp(r) - 1.0)


def _finish(part, bm):
    return pl.pallas_call(
        _finish_body,
        grid=(NG // bm,),
        in_specs=[pl.BlockSpec((NC, bm, ZC), lambda i: (0, i, 0))],
        out_specs=pl.BlockSpec((bm, HD), lambda i: (i, 0)),
        out_shape=jax.ShapeDtypeStruct((NG, HD), jnp.float32),
    )(part)


def kernel(user_emb, group_emb, W, attn_l, attn_r, src, dst):
    src = src.astype(jnp.int32)
    dst = dst.astype(jnp.int32)
    w3 = W.reshape(IN_DIM, H, D)
    wl = jnp.einsum("ihd,hd->ih", w3, attn_l)
    wr = jnp.einsum("ihd,hd->ih", w3, attn_r)
    w_aug = jnp.concatenate(
        [W, wl, jnp.zeros((IN_DIM, ZC - HD - H), jnp.float32)], axis=1)
    wr_pad = jnp.concatenate(
        [wr, jnp.zeros((IN_DIM, ERC - H), jnp.float32)], axis=1)
    zext, er = _project2(user_emb, group_emb, w_aug, wr_pad, 5000)
    part = _edge_phase()(zext, er, src, dst)
    return _finish(part, 5000)
